# baseline (speedup 1.0000x reference)
import functools

import jax
import jax.numpy as jnp
from jax.experimental import pallas as pl
from jax.experimental.pallas import tpu as pltpu

N_GENE = 20000
N_MESH = 30000
H = 128
L = 2


def _mm_body(x_ref, w_ref, b_ref, o_ref):
    o_ref[...] = (
        jnp.dot(x_ref[...], w_ref[...], preferred_element_type=jnp.float32)
        + b_ref[...]
    )


def _matmul_bias(x, w, b, block_m=400):
    m, k = x.shape
    n = w.shape[1]
    return pl.pallas_call(
        _mm_body,
        grid=(m // block_m,),
        in_specs=[
            pl.BlockSpec((block_m, k), lambda i: (i, 0)),
            pl.BlockSpec((k, n), lambda i: (0, 0)),
            pl.BlockSpec((1, n), lambda i: (0, 0)),
        ],
        out_specs=pl.BlockSpec((block_m, n), lambda i: (i, 0)),
        out_shape=jax.ShapeDtypeStruct((m, n), jnp.float32),
    )(x, w, b.reshape(1, n))


def _bn_relu(x, g, b, eps=1e-5):
    mu = jnp.mean(x, axis=0)
    var = jnp.var(x, axis=0)
    return jax.nn.relu((x - mu) / jnp.sqrt(var + eps) * g + b)


def _gcn(x, ei, W, b, n):
    src, dst = ei[0], ei[1]
    h = _matmul_bias(x, W, jnp.zeros((H,), jnp.float32))
    deg = jnp.zeros((n,), h.dtype).at[dst].add(1.0)
    dis = jnp.where(deg > 0, 1.0 / jnp.sqrt(jnp.maximum(deg, 1e-12)), 0.0)
    w = dis[src] * dis[dst]
    out = jnp.zeros((n, W.shape[1]), h.dtype).at[dst].add(w[:, None] * h[src])
    return out + b


def _gat(x_src, x_dst, ei, Ws, Wd, a_s, a_d, bias, n_dst):
    hs = _matmul_bias(x_src, Ws, jnp.zeros((H,), jnp.float32))
    hd = _matmul_bias(x_dst, Wd, jnp.zeros((H,), jnp.float32))
    es = jnp.sum(hs * a_s, axis=-1)
    ed = jnp.sum(hd * a_d, axis=-1)
    src, dst = ei[0], ei[1]
    e = jax.nn.leaky_relu(es[src] + ed[dst], 0.2)
    m = jax.ops.segment_max(e, dst, num_segments=n_dst)
    m = jnp.where(jnp.isfinite(m), m, 0.0)
    ex = jnp.exp(e - m[dst])
    den = jax.ops.segment_sum(ex, dst, num_segments=n_dst)
    alpha = ex / (den[dst] + 1e-16)
    out = jax.ops.segment_sum(alpha[:, None] * hs[src], dst, num_segments=n_dst)
    return out + bias


def kernel(x_gene, x_mesh, ei_ppi, ei_mm, ei_gm, ei_mg, p):
    gx = _bn_relu(_matmul_bias(x_gene, p['eg_W'], p['eg_b']), p['eg_g'], p['eg_be'])
    mx = _bn_relu(_matmul_bias(x_mesh, p['em_W'], p['em_b']), p['em_g'], p['em_be'])
    for l in range(L):
        mx = jax.nn.relu(_gcn(mx, ei_mm, p['mc_W'][l], p['mc_b'][l], N_MESH))
    for l in range(L):
        gx = jax.nn.relu(_gcn(gx, ei_ppi, p['gc_W'][l], p['gc_b'][l], N_GENE))
    for l in range(L):
        mx_new = jax.nn.relu(_gat(gx, mx, ei_gm, p['gm_Ws'][l], p['gm_Wd'][l], p['gm_as'][l], p['gm_ad'][l], p['gm_bi'][l], N_MESH))
        gx_new = jax.nn.relu(_gat(mx, gx, ei_mg, p['mg_Ws'][l], p['mg_Wd'][l], p['mg_as'][l], p['mg_ad'][l], p['mg_bi'][l], N_GENE))
        gx, mx = gx_new, mx_new
    gx = _matmul_bias(gx, p['lg_W'], p['lg_b'])
    mx = _matmul_bias(mx, p['lm_W'], p['lm_b'])
    return (gx, mx)
